# baseline (speedup 1.0000x reference)
import functools

import jax
import jax.numpy as jnp
from jax import lax
from jax.experimental import pallas as pl
from jax.experimental.pallas import tpu as pltpu
from jax.experimental.pallas import tpu_sc as plsc

N_CLUSTERS = 512
D = 32
N_TOKENS = 16 * 4096


_BS = 2048
_GRID = N_TOKENS // _BS


def _argmin_body(x_ref, c_ref, idx_ref):
    x = x_ref[...]
    c = c_ref[...]
    xc_t = lax.dot_general(
        c, x, (((1,), (1,)), ((), ())),
        preferred_element_type=jnp.float32,
        precision=lax.Precision.DEFAULT)
    hc2 = 0.5 * jnp.sum(c * c, axis=1)
    t = hc2[:, None] - xc_t
    idx_ref[...] = jnp.argmin(t, axis=0).astype(jnp.int32)


def _nearest_idx(xr, centers):
    return pl.pallas_call(
        _argmin_body,
        grid=(_GRID,),
        in_specs=[
            pl.BlockSpec((_BS, D), lambda i: (i, 0)),
            pl.BlockSpec((N_CLUSTERS, D), lambda i: (0, 0)),
        ],
        out_specs=pl.BlockSpec((_BS,), lambda i: (i,)),
        out_shape=jax.ShapeDtypeStruct((N_TOKENS,), jnp.int32),
    )(xr, centers)


_NC, _NS, _L = 2, 16, 16
_NW = _NC * _NS
_BPW = N_TOKENS // _NW
_CHT = 512
_NCHK = _BPW // _CHT
_NG = _CHT // _L


@functools.lru_cache(maxsize=None)
def _gather_fn():
    mesh = plsc.VectorSubcoreMesh(
        core_axis_name="c", subcore_axis_name="s",
        num_cores=_NC, num_subcores=_NS)

    @functools.partial(
        pl.kernel,
        mesh=mesh,
        out_type=jax.ShapeDtypeStruct((16, 4096, D), jnp.float32),
        scratch_types=[
            pltpu.VMEM((N_CLUSTERS * D,), jnp.float32),
            pltpu.VMEM((_BPW,), jnp.int32),
            pltpu.VMEM((_CHT, D), jnp.float32),
        ],
        compiler_params=pltpu.CompilerParams(
            use_tc_tiling_on_sc=True, needs_layout_passes=False),
    )
    def _gather_rows(table_hbm, idx_hbm, out_hbm, table_v, idx_v, rows_v):
        wid = lax.axis_index("s") * _NC + lax.axis_index("c")
        pltpu.sync_copy(table_hbm, table_v)
        pltpu.sync_copy(idx_hbm.at[pl.ds(wid * _BPW, _BPW)], idx_v)

        b = wid // 2
        s0 = (wid % 2) * _BPW
        tok = lax.iota(jnp.int32, _L)

        def chunk_body(k, carry):
            def body(g, carry2):
                idx16 = idx_v[pl.ds((k * _NG + g) * _L, _L)]
                src = idx16 * D
                row = g * _L + tok
                for d in range(D):
                    vals = plsc.load_gather(table_v, [src + d])
                    plsc.store_scatter(rows_v, [row, jnp.full((_L,), d, jnp.int32)], vals)
                return carry2

            lax.fori_loop(0, _NG, body, 0)
            pltpu.sync_copy(rows_v, out_hbm.at[b, pl.ds(s0 + k * _CHT, _CHT)])
            return carry

        lax.fori_loop(0, _NCHK, chunk_body, 0)

    return _gather_rows


def kernel(x, cluster_centers):
    xr = x.reshape(N_TOKENS, D)
    idx = _nearest_idx(xr, cluster_centers)
    table_flat = cluster_centers.reshape(N_CLUSTERS * D)
    return _gather_fn()(table_flat, idx)


# speedup vs baseline: 1.2007x; 1.2007x over previous
import functools

import jax
import jax.numpy as jnp
from jax import lax
from jax.experimental import pallas as pl
from jax.experimental.pallas import tpu as pltpu
from jax.experimental.pallas import tpu_sc as plsc

N_CLUSTERS = 512
D = 32
N_TOKENS = 16 * 4096


_BS = 2048
_GRID = N_TOKENS // _BS


def _argmin_body(x_ref, c_ref, idx_ref):
    x = x_ref[...]
    c = c_ref[...]
    xc_t = lax.dot_general(
        c, x, (((1,), (1,)), ((), ())),
        preferred_element_type=jnp.float32,
        precision=lax.Precision.DEFAULT)
    hc2 = 0.5 * jnp.sum(c * c, axis=1)
    t = hc2[:, None] - xc_t
    idx_ref[...] = jnp.argmin(t, axis=0).astype(jnp.int32)


def _nearest_idx(xr, centers):
    return pl.pallas_call(
        _argmin_body,
        grid=(_GRID,),
        in_specs=[
            pl.BlockSpec((_BS, D), lambda i: (i, 0)),
            pl.BlockSpec((N_CLUSTERS, D), lambda i: (0, 0)),
        ],
        out_specs=pl.BlockSpec((_BS,), lambda i: (i,)),
        out_shape=jax.ShapeDtypeStruct((N_TOKENS,), jnp.int32),
    )(xr, centers)


_NC, _NS, _L = 2, 16, 16
_NW = _NC * _NS
_BPW = N_TOKENS // _NW
_CHT = 512
_NCHK = _BPW // _CHT
_NG = _CHT // _L


@functools.lru_cache(maxsize=None)
def _gather_fn():
    mesh = plsc.VectorSubcoreMesh(
        core_axis_name="c", subcore_axis_name="s",
        num_cores=_NC, num_subcores=_NS)

    @functools.partial(
        pl.kernel,
        mesh=mesh,
        out_type=jax.ShapeDtypeStruct((16, 4096, D), jnp.float32),
        scratch_types=[
            pltpu.VMEM((N_CLUSTERS * D,), jnp.float32),
            pltpu.VMEM((_BPW,), jnp.int32),
            pltpu.VMEM((_CHT, D), jnp.float32),
        ],
        compiler_params=pltpu.CompilerParams(
            use_tc_tiling_on_sc=True, needs_layout_passes=False),
    )
    def _gather_rows(table_hbm, idx_hbm, out_hbm, table_v, idx_v, rows_v):
        wid = lax.axis_index("s") * _NC + lax.axis_index("c")
        pltpu.sync_copy(table_hbm, table_v)
        pltpu.sync_copy(idx_hbm.at[pl.ds(wid * _BPW, _BPW)], idx_v)

        b = wid // 2
        s0 = (wid % 2) * _BPW
        tok = lax.iota(jnp.int32, _L)

        for k in range(_NCHK):
            @plsc.parallel_loop(0, _NG)
            def body(g, _k=k):
                idx16 = idx_v[pl.ds((_k * _NG + g) * _L, _L)]
                src = idx16 * D
                row = g * _L + tok
                vals = [plsc.load_gather(table_v, [src + d]) for d in range(D)]
                for d in range(D):
                    plsc.store_scatter(
                        rows_v, [row, jnp.full((_L,), d, jnp.int32)], vals[d])

            pltpu.sync_copy(rows_v, out_hbm.at[b, pl.ds(s0 + k * _CHT, _CHT)])

    return _gather_rows


def kernel(x, cluster_centers):
    xr = x.reshape(N_TOKENS, D)
    idx = _nearest_idx(xr, cluster_centers)
    table_flat = cluster_centers.reshape(N_CLUSTERS * D)
    return _gather_fn()(table_flat, idx)


# speedup vs baseline: 1.7007x; 1.4165x over previous
import functools

import jax
import jax.numpy as jnp
from jax import lax
from jax.experimental import pallas as pl
from jax.experimental.pallas import tpu as pltpu
from jax.experimental.pallas import tpu_sc as plsc

N_CLUSTERS = 512
D = 32
N_TOKENS = 16 * 4096


_BS = 2048
_GRID = N_TOKENS // _BS


def _argmin_body(x_ref, c_ref, idx_ref):
    x = x_ref[...]
    c = c_ref[...]
    xc_t = lax.dot_general(
        c, x, (((1,), (1,)), ((), ())),
        preferred_element_type=jnp.float32,
        precision=lax.Precision.DEFAULT)
    hc2 = 0.5 * jnp.sum(c * c, axis=1)
    t = hc2[:, None] - xc_t
    idx_ref[...] = jnp.argmin(t, axis=0).astype(jnp.int32)


def _nearest_idx(xr, centers):
    return pl.pallas_call(
        _argmin_body,
        grid=(_GRID,),
        in_specs=[
            pl.BlockSpec((_BS, D), lambda i: (i, 0)),
            pl.BlockSpec((N_CLUSTERS, D), lambda i: (0, 0)),
        ],
        out_specs=pl.BlockSpec((_BS,), lambda i: (i,)),
        out_shape=jax.ShapeDtypeStruct((N_TOKENS,), jnp.int32),
    )(xr, centers)


_NC, _NS, _L = 2, 16, 16
_NW = _NC * _NS
_BPW = N_TOKENS // _NW
_CHT = 512
_NCHK = _BPW // _CHT
_NG = _CHT // _L


@functools.lru_cache(maxsize=None)
def _gather_fn():
    mesh = plsc.VectorSubcoreMesh(
        core_axis_name="c", subcore_axis_name="s",
        num_cores=_NC, num_subcores=_NS)

    @functools.partial(
        pl.kernel,
        mesh=mesh,
        out_type=jax.ShapeDtypeStruct((16, 4096, D), jnp.float32),
        scratch_types=[
            pltpu.VMEM((N_CLUSTERS * D,), jnp.float32),
            pltpu.VMEM((_BPW,), jnp.int32),
            pltpu.VMEM((_CHT, D), jnp.float32),
        ],
        compiler_params=pltpu.CompilerParams(
            use_tc_tiling_on_sc=True, needs_layout_passes=False),
    )
    def _gather_rows(table_hbm, idx_hbm, out_hbm, table_v, idx_v, rows_v):
        wid = lax.axis_index("s") * _NC + lax.axis_index("c")
        pltpu.sync_copy(table_hbm, table_v)
        pltpu.sync_copy(idx_hbm.at[pl.ds(wid * _BPW, _BPW)], idx_v)

        b = wid // 2
        s0 = (wid % 2) * _BPW
        lane = lax.iota(jnp.int32, _L)

        for k in range(_NCHK):
            @plsc.parallel_loop(0, _CHT, unroll=4)
            def body(t, _k=k):
                sidx = plsc.load_gather(
                    idx_v, [jnp.full((_L,), _k * _CHT, jnp.int32) + t])
                src = sidx * D + lane
                lo = plsc.load_gather(table_v, [src])
                hi = plsc.load_gather(table_v, [src + _L])
                rows_v[t, pl.ds(0, _L)] = lo
                rows_v[t, pl.ds(_L, _L)] = hi

            pltpu.sync_copy(rows_v, out_hbm.at[b, pl.ds(s0 + k * _CHT, _CHT)])

    return _gather_rows


def kernel(x, cluster_centers):
    xr = x.reshape(N_TOKENS, D)
    idx = _nearest_idx(xr, cluster_centers)
    table_flat = cluster_centers.reshape(N_CLUSTERS * D)
    return _gather_fn()(table_flat, idx)


# speedup vs baseline: 1.8514x; 1.0886x over previous
import functools

import jax
import jax.numpy as jnp
from jax import lax
from jax.experimental import pallas as pl
from jax.experimental.pallas import tpu as pltpu
from jax.experimental.pallas import tpu_sc as plsc

N_CLUSTERS = 512
D = 32
N_TOKENS = 16 * 4096


_BS = 4096
_GRID = N_TOKENS // _BS


def _argmin_body(x_ref, c_ref, idx_ref):
    x = x_ref[...]
    c = c_ref[...]
    xc_t = lax.dot_general(
        c, x, (((1,), (1,)), ((), ())),
        preferred_element_type=jnp.float32,
        precision=lax.Precision.DEFAULT)
    hc2 = 0.5 * jnp.sum(c * c, axis=1)
    t = hc2[:, None] - xc_t
    idx_ref[...] = jnp.argmin(t, axis=0).astype(jnp.int32)


def _nearest_idx(xr, centers):
    return pl.pallas_call(
        _argmin_body,
        grid=(_GRID,),
        in_specs=[
            pl.BlockSpec((_BS, D), lambda i: (i, 0)),
            pl.BlockSpec((N_CLUSTERS, D), lambda i: (0, 0)),
        ],
        out_specs=pl.BlockSpec((_BS,), lambda i: (i,)),
        out_shape=jax.ShapeDtypeStruct((N_TOKENS,), jnp.int32),
    )(xr, centers)


_NC, _NS, _L = 2, 16, 16
_NW = _NC * _NS
_BPW = N_TOKENS // _NW
_CHT = 512
_NCHK = _BPW // _CHT
_NG = _CHT // _L


@functools.lru_cache(maxsize=None)
def _gather_fn():
    mesh = plsc.VectorSubcoreMesh(
        core_axis_name="c", subcore_axis_name="s",
        num_cores=_NC, num_subcores=_NS)

    @functools.partial(
        pl.kernel,
        mesh=mesh,
        out_type=jax.ShapeDtypeStruct((16, 4096, D), jnp.float32),
        scratch_types=[
            pltpu.VMEM((N_CLUSTERS * D,), jnp.float32),
            pltpu.VMEM((_BPW,), jnp.int32),
            pltpu.VMEM((_CHT, D), jnp.float32),
        ],
        compiler_params=pltpu.CompilerParams(
            use_tc_tiling_on_sc=True, needs_layout_passes=False),
    )
    def _gather_rows(table_hbm, idx_hbm, out_hbm, table_v, idx_v, rows_v):
        wid = lax.axis_index("s") * _NC + lax.axis_index("c")
        pltpu.sync_copy(table_hbm, table_v)
        pltpu.sync_copy(idx_hbm.at[pl.ds(wid * _BPW, _BPW)], idx_v)

        b = wid // 2
        s0 = (wid % 2) * _BPW
        lane = lax.iota(jnp.int32, _L)

        for k in range(_NCHK):
            @plsc.parallel_loop(0, _CHT, unroll=4)
            def body(t, _k=k):
                sidx = plsc.load_gather(
                    idx_v, [jnp.full((_L,), _k * _CHT, jnp.int32) + t])
                src = sidx * D + lane
                lo = plsc.load_gather(table_v, [src])
                hi = plsc.load_gather(table_v, [src + _L])
                rows_v[t, pl.ds(0, _L)] = lo
                rows_v[t, pl.ds(_L, _L)] = hi

            pltpu.sync_copy(rows_v, out_hbm.at[b, pl.ds(s0 + k * _CHT, _CHT)])

    return _gather_rows


def kernel(x, cluster_centers):
    xr = x.reshape(N_TOKENS, D)
    idx = _nearest_idx(xr, cluster_centers)
    table_flat = cluster_centers.reshape(N_CLUSTERS * D)
    return _gather_fn()(table_flat, idx)


# speedup vs baseline: 3.2176x; 1.7380x over previous
import functools

import jax
import jax.numpy as jnp
from jax import lax
from jax.experimental import pallas as pl
from jax.experimental.pallas import tpu as pltpu
from jax.experimental.pallas import tpu_sc as plsc

N_CLUSTERS = 512
D = 32
B = 16
S = 4096
N_TOKENS = B * S


def _argmin_body(xt_ref, c_ref, idx_ref):
    xb = xt_ref[0]
    c = c_ref[...]
    xc_t = lax.dot_general(
        c, xb, (((1,), (0,)), ((), ())),
        preferred_element_type=jnp.float32,
        precision=lax.Precision.DEFAULT)
    hc2 = 0.5 * jnp.sum(c * c, axis=1)
    t = hc2[:, None] - xc_t
    idx_ref[...] = jnp.argmin(t, axis=0).astype(jnp.int32)


def _nearest_idx(xt, centers):
    return pl.pallas_call(
        _argmin_body,
        grid=(B,),
        in_specs=[
            pl.BlockSpec((1, D, S), lambda i: (i, 0, 0)),
            pl.BlockSpec((N_CLUSTERS, D), lambda i: (0, 0)),
        ],
        out_specs=pl.BlockSpec((S,), lambda i: (i,)),
        out_shape=jax.ShapeDtypeStruct((N_TOKENS,), jnp.int32),
    )(xt, centers)


_NC, _NS, _L = 2, 16, 16
_NW = _NC * _NS
_BPW = N_TOKENS // _NW
_NG = _BPW // _L


@functools.lru_cache(maxsize=None)
def _gather_fn():
    mesh = plsc.VectorSubcoreMesh(
        core_axis_name="c", subcore_axis_name="s",
        num_cores=_NC, num_subcores=_NS)

    @functools.partial(
        pl.kernel,
        mesh=mesh,
        out_type=jax.ShapeDtypeStruct((B, D, S), jnp.float32),
        scratch_types=[
            pltpu.VMEM((N_CLUSTERS * D,), jnp.float32),
            pltpu.VMEM((_BPW,), jnp.int32),
            pltpu.VMEM((D, _BPW), jnp.float32),
        ],
        compiler_params=pltpu.CompilerParams(
            use_tc_tiling_on_sc=True, needs_layout_passes=False),
    )
    def _gather_rows(tablet_hbm, idx_hbm, out_hbm, tablet_v, idx_v, rows_v):
        wid = lax.axis_index("s") * _NC + lax.axis_index("c")
        pltpu.sync_copy(tablet_hbm, tablet_v)
        pltpu.sync_copy(idx_hbm.at[pl.ds(wid * _BPW, _BPW)], idx_v)

        b = wid // 2
        s0 = (wid % 2) * _BPW

        @plsc.parallel_loop(0, _NG, unroll=2)
        def body(g):
            idx16 = idx_v[pl.ds(g * _L, _L)]
            for d in range(D):
                vals = plsc.load_gather(tablet_v, [idx16 + (d * N_CLUSTERS)])
                rows_v[d, pl.ds(g * _L, _L)] = vals

        pltpu.sync_copy(
            rows_v, out_hbm.at[b, pl.ds(0, D), pl.ds(s0, _BPW)])

    return _gather_rows


def kernel(x, cluster_centers):
    xt = jnp.swapaxes(x, 1, 2)
    idx = _nearest_idx(xt, cluster_centers)
    tablet = jnp.swapaxes(cluster_centers, 0, 1).reshape(N_CLUSTERS * D)
    outt = _gather_fn()(tablet, idx)
    return jnp.swapaxes(outt, 1, 2)


# speedup vs baseline: 3.2409x; 1.0072x over previous
import functools

import jax
import jax.numpy as jnp
from jax import lax
from jax.experimental import pallas as pl
from jax.experimental.pallas import tpu as pltpu
from jax.experimental.pallas import tpu_sc as plsc

N_CLUSTERS = 512
D = 32
B = 16
S = 4096
N_TOKENS = B * S


def _argmin_body(xt_ref, ct_ref, idx_ref):
    xb = xt_ref[0]
    ct = ct_ref[...]
    xc_t = lax.dot_general(
        ct, xb, (((0,), (0,)), ((), ())),
        preferred_element_type=jnp.float32,
        precision=lax.Precision.DEFAULT)
    hc2 = 0.5 * jnp.sum(ct * ct, axis=0)
    t = hc2[:, None] - xc_t
    idx_ref[...] = jnp.argmin(t, axis=0).astype(jnp.int32)


def _nearest_idx(xt, ct):
    return pl.pallas_call(
        _argmin_body,
        grid=(B,),
        in_specs=[
            pl.BlockSpec((1, D, S), lambda i: (i, 0, 0)),
            pl.BlockSpec((D, N_CLUSTERS), lambda i: (0, 0)),
        ],
        out_specs=pl.BlockSpec((S,), lambda i: (i,)),
        out_shape=jax.ShapeDtypeStruct((N_TOKENS,), jnp.int32),
    )(xt, ct)


_NC, _NS, _L = 2, 16, 16
_NW = _NC * _NS
_BPW = N_TOKENS // _NW
_NG = _BPW // _L


@functools.lru_cache(maxsize=None)
def _gather_fn():
    mesh = plsc.VectorSubcoreMesh(
        core_axis_name="c", subcore_axis_name="s",
        num_cores=_NC, num_subcores=_NS)

    @functools.partial(
        pl.kernel,
        mesh=mesh,
        out_type=jax.ShapeDtypeStruct((B, D, S), jnp.float32),
        scratch_types=[
            pltpu.VMEM((N_CLUSTERS * D,), jnp.float32),
            pltpu.VMEM((_BPW,), jnp.int32),
            pltpu.VMEM((D, _BPW), jnp.float32),
            pltpu.SemaphoreType.DMA,
        ],
        compiler_params=pltpu.CompilerParams(
            use_tc_tiling_on_sc=True, needs_layout_passes=False),
    )
    def _gather_rows(tablet_hbm, idx_hbm, out_hbm, tablet_v, idx_v, rows_v,
                     sem):
        wid = lax.axis_index("s") * _NC + lax.axis_index("c")
        pltpu.sync_copy(tablet_hbm, tablet_v)
        pltpu.sync_copy(idx_hbm.at[pl.ds(wid * _BPW, _BPW)], idx_v)

        b = wid // 2
        s0 = (wid % 2) * _BPW
        half = _BPW // 2

        copies = []
        for h in range(2):
            @plsc.parallel_loop(h * _NG // 2, (h + 1) * _NG // 2, unroll=2)
            def body(g):
                idx16 = idx_v[pl.ds(g * _L, _L)]
                for d in range(D):
                    vals = plsc.load_gather(
                        tablet_v, [idx16 + (d * N_CLUSTERS)])
                    rows_v[d, pl.ds(g * _L, _L)] = vals

            copies.append(pltpu.async_copy(
                rows_v.at[pl.ds(0, D), pl.ds(h * half, half)],
                out_hbm.at[b, pl.ds(0, D), pl.ds(s0 + h * half, half)],
                sem))
        for cp in copies:
            cp.wait()

    return _gather_rows


def kernel(x, cluster_centers):
    xt = jnp.swapaxes(x, 1, 2)
    ct = jnp.swapaxes(cluster_centers, 0, 1)
    idx = _nearest_idx(xt, ct)
    outt = _gather_fn()(ct.reshape(N_CLUSTERS * D), idx)
    return jnp.swapaxes(outt, 1, 2)
